# baseline (speedup 1.0000x reference)
import jax
import jax.numpy as jnp
from jax import lax
from jax.experimental import pallas as pl
from jax.experimental.pallas import tpu as pltpu
from jax.experimental.pallas import tpu_sc as plsc

_N = 10000
_E = 160000
_NCORES = 2
_NTILES = 16
_NACC = 10016
_IPT = _NACC // _NTILES
_OPT = _N // _NTILES


def _make_segsum(width, nch, ch, with_counts):
  out_types = [jax.ShapeDtypeStruct((_NCORES, _N, width), jnp.float32)]
  scratch = [
      pltpu.VMEM_SHARED((_NACC, width), jnp.float32),
      pltpu.VMEM((nch, ch), jnp.int32),
      pltpu.VMEM((nch, ch), jnp.int32),
      pltpu.VMEM((ch, width), jnp.float32),
      pltpu.VMEM((ch, width), jnp.float32),
      pltpu.SemaphoreType.DMA,
      pltpu.SemaphoreType.DMA,
  ]
  if with_counts:
    out_types.append(jax.ShapeDtypeStruct((_N, 16), jnp.float32))
    scratch += [
        pltpu.VMEM_SHARED((_NACC, 16), jnp.float32),
        pltpu.VMEM((ch, 16), jnp.float32),
    ]
  mesh = plsc.VectorSubcoreMesh(core_axis_name="c", subcore_axis_name="s")

  def body(*refs):
    it = iter(refs)
    table = next(it)
    src_hbm = next(it)
    dst_hbm = next(it)
    zeros_hbm = next(it)
    if with_counts:
      zcnt_hbm = next(it)
      ones_hbm = next(it)
    msg_hbm = next(it)
    if with_counts:
      cnt_hbm = next(it)
    acc_sh = next(it)
    src_v = next(it)
    dst_v = next(it)
    rows = (next(it), next(it))
    sems = (next(it), next(it))
    if with_counts:
      cnt_sh = next(it)
      ones_v = next(it)

    c = lax.axis_index("c")
    s = lax.axis_index("s")

    pltpu.sync_copy(zeros_hbm, acc_sh.at[pl.ds(s * _IPT, _IPT)])
    if with_counts:
      @pl.when(c == 0)
      def _():
        pltpu.sync_copy(zcnt_hbm, cnt_sh.at[pl.ds(s * _IPT, _IPT)])
      pltpu.sync_copy(ones_hbm, ones_v)
    pltpu.sync_copy(src_hbm.at[c, s], src_v)
    pltpu.sync_copy(dst_hbm.at[c, s], dst_v)
    plsc.subcore_barrier()

    def start_gather(j, b):
      pltpu.async_copy(table.at[src_v.at[j]], rows[b], sems[b])

    def consume(j, b):
      pltpu.make_async_copy(table.at[src_v.at[j]], rows[b], sems[b]).wait()
      nxt = j + 2

      @pl.when(nxt < nch)
      def _():
        start_gather(nxt, b)

      pltpu.sync_copy(rows[b], acc_sh.at[dst_v.at[j]], add=True)
      if with_counts:
        @pl.when(c == 0)
        def _():
          pltpu.sync_copy(ones_v, cnt_sh.at[dst_v.at[j]], add=True)

    start_gather(0, 0)
    start_gather(1, 1)

    @pl.loop(0, nch, step=2)
    def _(k):
      for b in range(2):
        consume(k + b, b)

    plsc.subcore_barrier()
    pltpu.sync_copy(acc_sh.at[pl.ds(s * _OPT, _OPT)],
                    msg_hbm.at[c, pl.ds(s * _OPT, _OPT)])
    if with_counts:
      @pl.when(c == 0)
      def _():
        pltpu.sync_copy(cnt_sh.at[pl.ds(s * _OPT, _OPT)],
                        cnt_hbm.at[pl.ds(s * _OPT, _OPT)])

  return pl.kernel(body, out_type=tuple(out_types), mesh=mesh,
                   scratch_types=scratch,
                   compiler_params=pltpu.CompilerParams(
                       use_tc_tiling_on_sc=False))


_NCH0, _CH0 = 256, 40
_NCH1, _CH1 = 128, 40
_EPAD0 = _NTILES * _NCH0 * _CH0
_EPAD1 = _NTILES * _NCH1 * _CH1
_segsum0 = _make_segsum(width=128, nch=_NCH0, ch=_CH0, with_counts=True)
_segsum1 = _make_segsum(width=48, nch=_NCH1, ch=_CH1, with_counts=False)

_RB = 1000


def _dense_xw_body(x_ref, wr_ref, b0_ref, xw_ref):
  xw_ref[...] = jnp.dot(x_ref[...], wr_ref[...],
                        preferred_element_type=jnp.float32) + b0_ref[...]


_dense_xw = pl.pallas_call(
    _dense_xw_body,
    grid=(_N // _RB,),
    in_specs=[
        pl.BlockSpec((_RB, 256), lambda i: (i, 0)),
        pl.BlockSpec((256, 256), lambda i: (0, 0)),
        pl.BlockSpec((1, 256), lambda i: (0, 0)),
    ],
    out_specs=pl.BlockSpec((_RB, 256), lambda i: (i, 0)),
    out_shape=jax.ShapeDtypeStruct((_N, 256), jnp.float32),
)


def _dense0_body(msg_ref, cnt_ref, xw_ref, wa_ref, w2a_ref, w2b_ref,
                 y1_ref, y2_ref):
  acc = jnp.dot(msg_ref[0], wa_ref[:128, :], preferred_element_type=jnp.float32)
  acc = acc + jnp.dot(msg_ref[1], wa_ref[128:, :],
                      preferred_element_type=jnp.float32)
  cnt = jnp.max(cnt_ref[...], axis=1, keepdims=True)
  inv = 1.0 / jnp.maximum(cnt, 1.0)
  x1 = jnp.maximum(acc * inv + xw_ref[...], 0.0)
  y1_ref[...] = jnp.dot(x1, w2a_ref[...], preferred_element_type=jnp.float32)
  y2_ref[...] = jnp.dot(x1, w2b_ref[...], preferred_element_type=jnp.float32)


_dense0 = pl.pallas_call(
    _dense0_body,
    grid=(_N // _RB,),
    in_specs=[
        pl.BlockSpec((_NCORES, _RB, 128), lambda i: (0, i, 0)),
        pl.BlockSpec((_RB, 16), lambda i: (i, 0)),
        pl.BlockSpec((_RB, 256), lambda i: (i, 0)),
        pl.BlockSpec((256, 256), lambda i: (0, 0)),
        pl.BlockSpec((256, 48), lambda i: (0, 0)),
        pl.BlockSpec((256, 48), lambda i: (0, 0)),
    ],
    out_specs=[
        pl.BlockSpec((_RB, 48), lambda i: (i, 0)),
        pl.BlockSpec((_RB, 48), lambda i: (i, 0)),
    ],
    out_shape=[
        jax.ShapeDtypeStruct((_N, 48), jnp.float32),
        jax.ShapeDtypeStruct((_N, 48), jnp.float32),
    ],
)


def _dense1_body(msg_ref, cnt_ref, y2_ref, c1_ref, out_ref):
  ssum = msg_ref[0] + msg_ref[1]
  cnt = jnp.max(cnt_ref[...], axis=1, keepdims=True)
  inv = 1.0 / jnp.maximum(cnt, 1.0)
  res = ssum * inv + y2_ref[...] + c1_ref[...]
  out_ref[...] = res[:, :40]


_dense1 = pl.pallas_call(
    _dense1_body,
    grid=(_N // _RB,),
    in_specs=[
        pl.BlockSpec((_NCORES, _RB, 48), lambda i: (0, i, 0)),
        pl.BlockSpec((_RB, 16), lambda i: (i, 0)),
        pl.BlockSpec((_RB, 48), lambda i: (i, 0)),
        pl.BlockSpec((1, 48), lambda i: (0, 0)),
    ],
    out_specs=pl.BlockSpec((_RB, 40), lambda i: (i, 0)),
    out_shape=jax.ShapeDtypeStruct((_N, 40), jnp.float32),
)


def kernel(x, edge_index, sage0_Wl, sage0_bl, sage0_Wr, lin0_W, lin0_b,
           sage1_Wl, sage1_bl, sage1_Wr, lin1_W, lin1_b):
  src = edge_index[0].astype(jnp.int32)
  dst = edge_index[1].astype(jnp.int32)

  x2 = x.reshape(2 * _N, 128)
  p0 = (_EPAD0 - _E) // _NTILES
  dum0 = jnp.broadcast_to(_N + (jnp.arange(p0, dtype=jnp.int32) % 16),
                          (_NTILES, p0))
  src0 = jnp.concatenate(
      [src.reshape(_NTILES, -1) * 2,
       jnp.zeros((_NTILES, p0), jnp.int32)], axis=1)
  dst0 = jnp.concatenate([dst.reshape(_NTILES, -1), dum0], axis=1)
  srcA = jnp.stack([src0, src0 + 1]).reshape(_NCORES, _NTILES, _NCH0, _CH0)
  dstA = jnp.broadcast_to(
      dst0.reshape(1, _NTILES, _NCH0, _CH0), (_NCORES, _NTILES, _NCH0, _CH0))
  zeros128 = jnp.zeros((_IPT, 128), jnp.float32)
  zeros16 = jnp.zeros((_IPT, 16), jnp.float32)
  ones = jnp.ones((_CH0, 16), jnp.float32)
  msg0, cnt = _segsum0(x2, srcA, dstA, zeros128, zeros16, ones)

  b0 = (sage0_bl + lin0_b).reshape(1, 256)
  xw = _dense_xw(x, sage0_Wr.T, b0)
  w2a = jnp.pad(sage1_Wl.T, ((0, 0), (0, 8)))
  w2b = jnp.pad(sage1_Wr.T, ((0, 0), (0, 8)))
  y1, y2 = _dense0(msg0, cnt, xw, sage0_Wl.T, w2a, w2b)

  p1 = (_EPAD1 - _E // 2) // _NTILES
  dum1 = jnp.broadcast_to(_N + (jnp.arange(p1, dtype=jnp.int32) % 16),
                          (_NCORES, _NTILES, p1))
  srcC = jnp.concatenate(
      [src.reshape(_NCORES, _NTILES, -1),
       jnp.zeros((_NCORES, _NTILES, p1), jnp.int32)], axis=2).reshape(
          _NCORES, _NTILES, _NCH1, _CH1)
  dstC = jnp.concatenate(
      [dst.reshape(_NCORES, _NTILES, -1), dum1], axis=2).reshape(
          _NCORES, _NTILES, _NCH1, _CH1)
  zeros48 = jnp.zeros((_IPT, 48), jnp.float32)
  (msg1,) = _segsum1(y1, srcC, dstC, zeros48)

  c1 = (sage1_bl + lin1_W @ lin0_b + lin1_b)
  c1p = jnp.pad(c1, (0, 8)).reshape(1, 48)
  return _dense1(msg1, cnt, y2, c1p)


# speedup vs baseline: 1.7627x; 1.7627x over previous
import jax
import jax.numpy as jnp
from jax import lax
from jax.experimental import pallas as pl
from jax.experimental.pallas import tpu as pltpu
from jax.experimental.pallas import tpu_sc as plsc

_N = 10000
_E = 160000
_NCORES = 2
_NTILES = 16
_RPT = _N // _NTILES


def _make_segsum(width, nch, ch, with_counts, split_edges, two_tables):
  out_types = [jax.ShapeDtypeStruct((_NCORES, _N, width), jnp.float32)]
  scratch = [
      pltpu.VMEM_SHARED((_N, width), jnp.float32),
      pltpu.VMEM((nch, ch), jnp.int32),
      pltpu.VMEM((nch, ch), jnp.int32),
      pltpu.VMEM((ch, width), jnp.float32),
      pltpu.VMEM((ch, width), jnp.float32),
      pltpu.SemaphoreType.DMA,
      pltpu.SemaphoreType.DMA,
  ]
  if with_counts:
    out_types.append(jax.ShapeDtypeStruct((_N, 16), jnp.float32))
    scratch += [
        pltpu.VMEM_SHARED((_N, 16), jnp.float32),
        pltpu.VMEM((ch, 16), jnp.float32),
    ]
  mesh = plsc.VectorSubcoreMesh(core_axis_name="c", subcore_axis_name="s")
  n_tables = 2 if two_tables else 1

  def body(*refs):
    it = iter(refs)
    tables = [next(it) for _ in range(n_tables)]
    src_hbm = next(it)
    dst_hbm = next(it)
    zeros_hbm = next(it)
    if with_counts:
      zcnt_hbm = next(it)
      ones_hbm = next(it)
    msg_hbm = next(it)
    if with_counts:
      cnt_hbm = next(it)
    acc_sh = next(it)
    src_v = next(it)
    dst_v = next(it)
    rows = (next(it), next(it))
    sems = (next(it), next(it))
    if with_counts:
      cnt_sh = next(it)
      ones_v = next(it)

    c = lax.axis_index("c")
    s = lax.axis_index("s")

    pltpu.sync_copy(zeros_hbm, acc_sh.at[pl.ds(s * _RPT, _RPT)])
    if with_counts:
      @pl.when(c == 0)
      def _():
        pltpu.sync_copy(zcnt_hbm, cnt_sh.at[pl.ds(s * _RPT, _RPT)])
      pltpu.sync_copy(ones_hbm, ones_v)
    if split_edges:
      pltpu.sync_copy(src_hbm.at[c, s], src_v)
      pltpu.sync_copy(dst_hbm.at[c, s], dst_v)
    else:
      pltpu.sync_copy(src_hbm.at[s], src_v)
      pltpu.sync_copy(dst_hbm.at[s], dst_v)
    plsc.subcore_barrier()

    def start_gather(j, b):
      if two_tables:
        @pl.when(c == 0)
        def _():
          pltpu.async_copy(tables[0].at[src_v.at[j]], rows[b], sems[b])

        @pl.when(c == 1)
        def _():
          pltpu.async_copy(tables[1].at[src_v.at[j]], rows[b], sems[b])
      else:
        pltpu.async_copy(tables[0].at[src_v.at[j]], rows[b], sems[b])

    def consume(j, b):
      pltpu.make_async_copy(tables[0].at[src_v.at[j]], rows[b],
                            sems[b]).wait()
      nxt = j + 2

      @pl.when(nxt < nch)
      def _():
        start_gather(nxt, b)

      pltpu.sync_copy(rows[b], acc_sh.at[dst_v.at[j]], add=True)
      if with_counts:
        @pl.when(c == 0)
        def _():
          pltpu.sync_copy(ones_v, cnt_sh.at[dst_v.at[j]], add=True)

    start_gather(0, 0)
    if nch > 1:
      start_gather(1, 1)
    main = nch if nch % 2 == 0 else nch - 1

    @pl.loop(0, main, step=2)
    def _(k):
      for b in range(2):
        consume(k + b, b)

    if nch % 2 == 1:
      consume(nch - 1, 0)

    plsc.subcore_barrier()
    pltpu.sync_copy(acc_sh.at[pl.ds(s * _RPT, _RPT)],
                    msg_hbm.at[c, pl.ds(s * _RPT, _RPT)])
    if with_counts:
      @pl.when(c == 0)
      def _():
        pltpu.sync_copy(cnt_sh.at[pl.ds(s * _RPT, _RPT)],
                        cnt_hbm.at[pl.ds(s * _RPT, _RPT)])

  return pl.kernel(body, out_type=tuple(out_types), mesh=mesh,
                   scratch_types=scratch,
                   compiler_params=pltpu.CompilerParams(
                       use_tc_tiling_on_sc=False))


_segsum0 = _make_segsum(width=128, nch=250, ch=40, with_counts=True,
                        split_edges=False, two_tables=True)
_segsum1 = _make_segsum(width=48, nch=125, ch=40, with_counts=False,
                        split_edges=True, two_tables=False)

_RB = 1000


def _dense_xw_body(x_ref, wr_ref, b0_ref, xw_ref):
  xw_ref[...] = jnp.dot(x_ref[...], wr_ref[...],
                        preferred_element_type=jnp.float32) + b0_ref[...]


_dense_xw = pl.pallas_call(
    _dense_xw_body,
    grid=(_N // _RB,),
    in_specs=[
        pl.BlockSpec((_RB, 256), lambda i: (i, 0)),
        pl.BlockSpec((256, 256), lambda i: (0, 0)),
        pl.BlockSpec((1, 256), lambda i: (0, 0)),
    ],
    out_specs=pl.BlockSpec((_RB, 256), lambda i: (i, 0)),
    out_shape=jax.ShapeDtypeStruct((_N, 256), jnp.float32),
)


def _dense0_body(msg_ref, cnt_ref, xw_ref, wa_ref, w2a_ref, w2b_ref,
                 y1_ref, y2_ref):
  acc = jnp.dot(msg_ref[0], wa_ref[:128, :], preferred_element_type=jnp.float32)
  acc = acc + jnp.dot(msg_ref[1], wa_ref[128:, :],
                      preferred_element_type=jnp.float32)
  cnt = jnp.max(cnt_ref[...], axis=1, keepdims=True)
  inv = 1.0 / jnp.maximum(cnt, 1.0)
  x1 = jnp.maximum(acc * inv + xw_ref[...], 0.0)
  y1_ref[...] = jnp.dot(x1, w2a_ref[...], preferred_element_type=jnp.float32)
  y2_ref[...] = jnp.dot(x1, w2b_ref[...], preferred_element_type=jnp.float32)


_dense0 = pl.pallas_call(
    _dense0_body,
    grid=(_N // _RB,),
    in_specs=[
        pl.BlockSpec((_NCORES, _RB, 128), lambda i: (0, i, 0)),
        pl.BlockSpec((_RB, 16), lambda i: (i, 0)),
        pl.BlockSpec((_RB, 256), lambda i: (i, 0)),
        pl.BlockSpec((256, 256), lambda i: (0, 0)),
        pl.BlockSpec((256, 48), lambda i: (0, 0)),
        pl.BlockSpec((256, 48), lambda i: (0, 0)),
    ],
    out_specs=[
        pl.BlockSpec((_RB, 48), lambda i: (i, 0)),
        pl.BlockSpec((_RB, 48), lambda i: (i, 0)),
    ],
    out_shape=[
        jax.ShapeDtypeStruct((_N, 48), jnp.float32),
        jax.ShapeDtypeStruct((_N, 48), jnp.float32),
    ],
)


def _dense1_body(msg_ref, cnt_ref, y2_ref, c1_ref, out_ref):
  ssum = msg_ref[0] + msg_ref[1]
  cnt = jnp.max(cnt_ref[...], axis=1, keepdims=True)
  inv = 1.0 / jnp.maximum(cnt, 1.0)
  res = ssum * inv + y2_ref[...] + c1_ref[...]
  out_ref[...] = res[:, :40]


_dense1 = pl.pallas_call(
    _dense1_body,
    grid=(_N // _RB,),
    in_specs=[
        pl.BlockSpec((_NCORES, _RB, 48), lambda i: (0, i, 0)),
        pl.BlockSpec((_RB, 16), lambda i: (i, 0)),
        pl.BlockSpec((_RB, 48), lambda i: (i, 0)),
        pl.BlockSpec((1, 48), lambda i: (0, 0)),
    ],
    out_specs=pl.BlockSpec((_RB, 40), lambda i: (i, 0)),
    out_shape=jax.ShapeDtypeStruct((_N, 40), jnp.float32),
)


def kernel(x, edge_index, sage0_Wl, sage0_bl, sage0_Wr, lin0_W, lin0_b,
           sage1_Wl, sage1_bl, sage1_Wr, lin1_W, lin1_b):
  src = edge_index[0].astype(jnp.int32)
  dst = edge_index[1].astype(jnp.int32)

  x_lo = x[:, :128]
  x_hi = x[:, 128:]
  srcA = src.reshape(_NTILES, 250, 40)
  dstA = dst.reshape(_NTILES, 250, 40)
  zeros128 = jnp.zeros((_RPT, 128), jnp.float32)
  zeros16 = jnp.zeros((_RPT, 16), jnp.float32)
  ones = jnp.ones((40, 16), jnp.float32)
  msg0, cnt = _segsum0(x_lo, x_hi, srcA, dstA, zeros128, zeros16, ones)

  b0 = (sage0_bl + lin0_b).reshape(1, 256)
  xw = _dense_xw(x, sage0_Wr.T, b0)
  w2a = jnp.pad(sage1_Wl.T, ((0, 0), (0, 8)))
  w2b = jnp.pad(sage1_Wr.T, ((0, 0), (0, 8)))
  y1, y2 = _dense0(msg0, cnt, xw, sage0_Wl.T, w2a, w2b)

  srcC = src.reshape(_NCORES, _NTILES, 125, 40)
  dstC = dst.reshape(_NCORES, _NTILES, 125, 40)
  zeros48 = jnp.zeros((_RPT, 48), jnp.float32)
  (msg1,) = _segsum1(y1, srcC, dstC, zeros48)

  c1 = (sage1_bl + lin1_W @ lin0_b + lin1_b)
  c1p = jnp.pad(c1, (0, 8)).reshape(1, 48)
  return _dense1(msg1, cnt, y2, c1p)


# speedup vs baseline: 1.7715x; 1.0050x over previous
import jax
import jax.numpy as jnp
from jax import lax
from jax.experimental import pallas as pl
from jax.experimental.pallas import tpu as pltpu
from jax.experimental.pallas import tpu_sc as plsc

_N = 10000
_E = 160000
_NCORES = 2
_NTILES = 16
_OPT = _N // _NTILES


def _make_segsum(width, nch, ch, nbuf, nacc, with_counts, split_edges,
                 two_tables):
  ipt = nacc // _NTILES
  out_types = [jax.ShapeDtypeStruct((_NCORES, _N, width), jnp.float32)]
  scratch = [
      pltpu.VMEM_SHARED((nacc, width), jnp.float32),
      pltpu.VMEM((nch, ch), jnp.int32),
      pltpu.VMEM((nch, ch), jnp.int32),
  ] + [pltpu.VMEM((ch, width), jnp.float32) for _ in range(nbuf)] + [
      pltpu.SemaphoreType.DMA for _ in range(nbuf)
  ]
  if with_counts:
    out_types.append(jax.ShapeDtypeStruct((_NCORES, _N, 16), jnp.float32))
    scratch += [
        pltpu.VMEM_SHARED((nacc, 16), jnp.float32),
        pltpu.VMEM((ch, 16), jnp.float32),
    ]
  mesh = plsc.VectorSubcoreMesh(core_axis_name="c", subcore_axis_name="s")
  n_tables = 2 if two_tables else 1

  def body(*refs):
    it = iter(refs)
    tables = [next(it) for _ in range(n_tables)]
    src_hbm = next(it)
    dst_hbm = next(it)
    zeros_hbm = next(it)
    if with_counts:
      zcnt_hbm = next(it)
      ones_hbm = next(it)
    msg_hbm = next(it)
    if with_counts:
      cnt_hbm = next(it)
    acc_sh = next(it)
    src_v = next(it)
    dst_v = next(it)
    rows = tuple(next(it) for _ in range(nbuf))
    sems = tuple(next(it) for _ in range(nbuf))
    if with_counts:
      cnt_sh = next(it)
      ones_v = next(it)

    c = lax.axis_index("c")
    s = lax.axis_index("s")

    pltpu.sync_copy(zeros_hbm, acc_sh.at[pl.ds(s * ipt, ipt)])
    if with_counts:
      pltpu.sync_copy(zcnt_hbm, cnt_sh.at[pl.ds(s * ipt, ipt)])
      pltpu.sync_copy(ones_hbm, ones_v)
    if split_edges:
      pltpu.sync_copy(src_hbm.at[c, s], src_v)
      pltpu.sync_copy(dst_hbm.at[c, s], dst_v)
    else:
      pltpu.sync_copy(src_hbm.at[s], src_v)
      pltpu.sync_copy(dst_hbm.at[s], dst_v)
    plsc.subcore_barrier()

    def start_gather(j, b):
      if two_tables:
        @pl.when(c == 0)
        def _():
          pltpu.async_copy(tables[0].at[src_v.at[j]], rows[b], sems[b])

        @pl.when(c == 1)
        def _():
          pltpu.async_copy(tables[1].at[src_v.at[j]], rows[b], sems[b])
      else:
        pltpu.async_copy(tables[0].at[src_v.at[j]], rows[b], sems[b])

    def consume(j, b):
      pltpu.make_async_copy(tables[0].at[src_v.at[j]], rows[b],
                            sems[b]).wait()
      nxt = j + nbuf

      @pl.when(nxt < nch)
      def _():
        start_gather(nxt, b)

      pltpu.sync_copy(rows[b], acc_sh.at[dst_v.at[j]], add=True)
      if with_counts:
        @pl.when(((c == 0) & (j < nch // 2)) | ((c == 1) & (j >= nch // 2)))
        def _():
          pltpu.sync_copy(ones_v, cnt_sh.at[dst_v.at[j]], add=True)

    for b in range(min(nbuf, nch)):
      start_gather(b, b)
    main = nch - nch % nbuf

    @pl.loop(0, main, step=nbuf)
    def _(k):
      for b in range(nbuf):
        consume(k + b, b)

    for r in range(nch % nbuf):
      consume(main + r, r)

    plsc.subcore_barrier()
    pltpu.sync_copy(acc_sh.at[pl.ds(s * _OPT, _OPT)],
                    msg_hbm.at[c, pl.ds(s * _OPT, _OPT)])
    if with_counts:
      pltpu.sync_copy(cnt_sh.at[pl.ds(s * _OPT, _OPT)],
                      cnt_hbm.at[c, pl.ds(s * _OPT, _OPT)])

  return pl.kernel(body, out_type=tuple(out_types), mesh=mesh,
                   scratch_types=scratch,
                   compiler_params=pltpu.CompilerParams(
                       use_tc_tiling_on_sc=False))


_NCH0, _CH0 = 250, 40
_NCH1, _CH1 = 40, 128
_NACC1 = _N + 16 * _NTILES
_EPT1 = _E // _NCORES // _NTILES
_DPT1 = _NCH1 * _CH1 - _EPT1
_segsum0 = _make_segsum(width=128, nch=_NCH0, ch=_CH0, nbuf=3, nacc=_N,
                        with_counts=True, split_edges=False, two_tables=True)
_segsum1 = _make_segsum(width=48, nch=_NCH1, ch=_CH1, nbuf=4, nacc=_NACC1,
                        with_counts=False, split_edges=True, two_tables=False)

_RB = 1000


def _dense0_body(msg_ref, cnt_ref, x_ref, wa_ref, wr_ref, b0_ref,
                 w2a_ref, w2b_ref, y1_ref, y2_ref):
  acc = jnp.dot(msg_ref[0], wa_ref[:128, :], preferred_element_type=jnp.float32)
  acc = acc + jnp.dot(msg_ref[1], wa_ref[128:, :],
                      preferred_element_type=jnp.float32)
  cnt = jnp.max(cnt_ref[0] + cnt_ref[1], axis=1, keepdims=True)
  inv = 1.0 / jnp.maximum(cnt, 1.0)
  h = acc * inv + jnp.dot(x_ref[...], wr_ref[...],
                          preferred_element_type=jnp.float32) + b0_ref[...]
  x1 = jnp.maximum(h, 0.0)
  y1_ref[...] = jnp.dot(x1, w2a_ref[...], preferred_element_type=jnp.float32)
  y2_ref[...] = jnp.dot(x1, w2b_ref[...], preferred_element_type=jnp.float32)


_dense0 = pl.pallas_call(
    _dense0_body,
    grid=(_N // _RB,),
    in_specs=[
        pl.BlockSpec((_NCORES, _RB, 128), lambda i: (0, i, 0)),
        pl.BlockSpec((_NCORES, _RB, 16), lambda i: (0, i, 0)),
        pl.BlockSpec((_RB, 256), lambda i: (i, 0)),
        pl.BlockSpec((256, 256), lambda i: (0, 0)),
        pl.BlockSpec((256, 256), lambda i: (0, 0)),
        pl.BlockSpec((1, 256), lambda i: (0, 0)),
        pl.BlockSpec((256, 48), lambda i: (0, 0)),
        pl.BlockSpec((256, 48), lambda i: (0, 0)),
    ],
    out_specs=[
        pl.BlockSpec((_RB, 48), lambda i: (i, 0)),
        pl.BlockSpec((_RB, 48), lambda i: (i, 0)),
    ],
    out_shape=[
        jax.ShapeDtypeStruct((_N, 48), jnp.float32),
        jax.ShapeDtypeStruct((_N, 48), jnp.float32),
    ],
)


def _dense1_body(msg_ref, cnt_ref, y2_ref, c1_ref, out_ref):
  ssum = msg_ref[0] + msg_ref[1]
  cnt = jnp.max(cnt_ref[0] + cnt_ref[1], axis=1, keepdims=True)
  inv = 1.0 / jnp.maximum(cnt, 1.0)
  res = ssum * inv + y2_ref[...] + c1_ref[...]
  out_ref[...] = res[:, :40]


_dense1 = pl.pallas_call(
    _dense1_body,
    grid=(_N // _RB,),
    in_specs=[
        pl.BlockSpec((_NCORES, _RB, 48), lambda i: (0, i, 0)),
        pl.BlockSpec((_NCORES, _RB, 16), lambda i: (0, i, 0)),
        pl.BlockSpec((_RB, 48), lambda i: (i, 0)),
        pl.BlockSpec((1, 48), lambda i: (0, 0)),
    ],
    out_specs=pl.BlockSpec((_RB, 40), lambda i: (i, 0)),
    out_shape=jax.ShapeDtypeStruct((_N, 40), jnp.float32),
)


def kernel(x, edge_index, sage0_Wl, sage0_bl, sage0_Wr, lin0_W, lin0_b,
           sage1_Wl, sage1_bl, sage1_Wr, lin1_W, lin1_b):
  src = edge_index[0].astype(jnp.int32)
  dst = edge_index[1].astype(jnp.int32)

  x_lo = x[:, :128]
  x_hi = x[:, 128:]
  srcA = src.reshape(_NTILES, _NCH0, _CH0)
  dstA = dst.reshape(_NTILES, _NCH0, _CH0)
  zeros128 = jnp.zeros((_N // _NTILES, 128), jnp.float32)
  zeros16 = jnp.zeros((_N // _NTILES, 16), jnp.float32)
  ones = jnp.ones((_CH0, 16), jnp.float32)
  msg0, cnt = _segsum0(x_lo, x_hi, srcA, dstA, zeros128, zeros16, ones)

  b0 = (sage0_bl + lin0_b).reshape(1, 256)
  w2a = jnp.pad(sage1_Wl.T, ((0, 0), (0, 8)))
  w2b = jnp.pad(sage1_Wr.T, ((0, 0), (0, 8)))
  y1, y2 = _dense0(msg0, cnt, x, sage0_Wl.T, sage0_Wr.T, b0, w2a, w2b)

  dum = (_N + 16 * jnp.arange(_NTILES, dtype=jnp.int32)[:, None]
         + (jnp.arange(_DPT1, dtype=jnp.int32) % 16)[None, :])
  dum = jnp.broadcast_to(dum[None], (_NCORES, _NTILES, _DPT1))
  srcC = jnp.concatenate(
      [src.reshape(_NCORES, _NTILES, _EPT1),
       jnp.zeros((_NCORES, _NTILES, _DPT1), jnp.int32)], axis=2).reshape(
           _NCORES, _NTILES, _NCH1, _CH1)
  dstC = jnp.concatenate(
      [dst.reshape(_NCORES, _NTILES, _EPT1), dum], axis=2).reshape(
          _NCORES, _NTILES, _NCH1, _CH1)
  zeros48 = jnp.zeros((_NACC1 // _NTILES, 48), jnp.float32)
  (msg1,) = _segsum1(y1, srcC, dstC, zeros48)

  c1 = (sage1_bl + lin1_W @ lin0_b + lin1_b)
  c1p = jnp.pad(c1, (0, 8)).reshape(1, 48)
  return _dense1(msg1, cnt, y2, c1p)


# speedup vs baseline: 2.1180x; 1.1956x over previous
import jax
import jax.numpy as jnp
from jax import lax
from jax.experimental import pallas as pl
from jax.experimental.pallas import tpu as pltpu
from jax.experimental.pallas import tpu_sc as plsc

_N = 10000
_E = 160000
_NCORES = 2
_NTILES = 16
_OPT = _N // _NTILES


def _make_segsum(width, nch, ch, nbuf, nacc, with_counts, split_edges,
                 two_tables):
  ipt = nacc // _NTILES
  out_types = [jax.ShapeDtypeStruct((_NCORES, _N, width), jnp.float32)]
  scratch = [
      pltpu.VMEM_SHARED((nacc, width), jnp.float32),
      pltpu.VMEM((nch, ch), jnp.int32),
      pltpu.VMEM((nch, ch), jnp.int32),
  ] + [pltpu.VMEM((ch, width), jnp.float32) for _ in range(nbuf)] + [
      pltpu.SemaphoreType.DMA for _ in range(nbuf)
  ]
  if with_counts:
    out_types.append(jax.ShapeDtypeStruct((_NCORES, _N, 16), jnp.float32))
    scratch += [
        pltpu.VMEM_SHARED((nacc, 16), jnp.float32),
        pltpu.VMEM((ch, 16), jnp.float32),
    ]
  mesh = plsc.VectorSubcoreMesh(core_axis_name="c", subcore_axis_name="s")
  n_tables = 2 if two_tables else 1

  def body(*refs):
    it = iter(refs)
    tables = [next(it) for _ in range(n_tables)]
    src_hbm = next(it)
    dst_hbm = next(it)
    zeros_hbm = next(it)
    if with_counts:
      zcnt_hbm = next(it)
      ones_hbm = next(it)
    msg_hbm = next(it)
    if with_counts:
      cnt_hbm = next(it)
    acc_sh = next(it)
    src_v = next(it)
    dst_v = next(it)
    rows = tuple(next(it) for _ in range(nbuf))
    sems = tuple(next(it) for _ in range(nbuf))
    if with_counts:
      cnt_sh = next(it)
      ones_v = next(it)

    c = lax.axis_index("c")
    s = lax.axis_index("s")

    pltpu.sync_copy(zeros_hbm, acc_sh.at[pl.ds(s * ipt, ipt)])
    if with_counts:
      pltpu.sync_copy(zcnt_hbm, cnt_sh.at[pl.ds(s * ipt, ipt)])
      pltpu.sync_copy(ones_hbm, ones_v)
    if split_edges:
      pltpu.sync_copy(src_hbm.at[c, s], src_v)
      pltpu.sync_copy(dst_hbm.at[c, s], dst_v)
    else:
      pltpu.sync_copy(src_hbm.at[s], src_v)
      pltpu.sync_copy(dst_hbm.at[s], dst_v)
    plsc.subcore_barrier()

    def start_gather(j, b):
      if two_tables:
        @pl.when(c == 0)
        def _():
          pltpu.async_copy(tables[0].at[src_v.at[j]], rows[b], sems[b])

        @pl.when(c == 1)
        def _():
          pltpu.async_copy(tables[1].at[src_v.at[j]], rows[b], sems[b])
      else:
        pltpu.async_copy(tables[0].at[src_v.at[j]], rows[b], sems[b])

    def consume(j, b):
      pltpu.make_async_copy(tables[0].at[src_v.at[j]], rows[b],
                            sems[b]).wait()
      nxt = j + nbuf

      @pl.when(nxt < nch)
      def _():
        start_gather(nxt, b)

      pltpu.sync_copy(rows[b], acc_sh.at[dst_v.at[j]], add=True)
      if with_counts:
        @pl.when(((c == 0) & (j < nch // 2)) | ((c == 1) & (j >= nch // 2)))
        def _():
          pltpu.sync_copy(ones_v, cnt_sh.at[dst_v.at[j]], add=True)

    for b in range(min(nbuf, nch)):
      start_gather(b, b)
    main = nch - nch % nbuf

    @pl.loop(0, main, step=nbuf)
    def _(k):
      for b in range(nbuf):
        consume(k + b, b)

    for r in range(nch % nbuf):
      consume(main + r, r)

    plsc.subcore_barrier()
    pltpu.sync_copy(acc_sh.at[pl.ds(s * _OPT, _OPT)],
                    msg_hbm.at[c, pl.ds(s * _OPT, _OPT)])
    if with_counts:
      pltpu.sync_copy(cnt_sh.at[pl.ds(s * _OPT, _OPT)],
                      cnt_hbm.at[c, pl.ds(s * _OPT, _OPT)])

  return pl.kernel(body, out_type=tuple(out_types), mesh=mesh,
                   scratch_types=scratch,
                   compiler_params=pltpu.CompilerParams(
                       use_tc_tiling_on_sc=False))


_NCH0, _CH0 = 250, 40
_NCH1, _CH1 = 125, 40
_segsum0 = _make_segsum(width=128, nch=_NCH0, ch=_CH0, nbuf=3, nacc=_N,
                        with_counts=True, split_edges=False, two_tables=True)
_segsum1 = _make_segsum(width=48, nch=_NCH1, ch=_CH1, nbuf=4, nacc=_N,
                        with_counts=False, split_edges=True, two_tables=False)

_RB = 1000


def _dense0_body(msg_ref, cnt_ref, x_ref, wa_ref, wr_ref, b0_ref,
                 w2a_ref, w2b_ref, y1_ref, y2_ref):
  acc = jnp.dot(msg_ref[0], wa_ref[:128, :], preferred_element_type=jnp.float32)
  acc = acc + jnp.dot(msg_ref[1], wa_ref[128:, :],
                      preferred_element_type=jnp.float32)
  cnt = jnp.max(cnt_ref[0] + cnt_ref[1], axis=1, keepdims=True)
  inv = 1.0 / jnp.maximum(cnt, 1.0)
  h = acc * inv + jnp.dot(x_ref[...], wr_ref[...],
                          preferred_element_type=jnp.float32) + b0_ref[...]
  x1 = jnp.maximum(h, 0.0)
  y1_ref[...] = jnp.dot(x1, w2a_ref[...], preferred_element_type=jnp.float32)
  y2_ref[...] = jnp.dot(x1, w2b_ref[...], preferred_element_type=jnp.float32)


_dense0 = pl.pallas_call(
    _dense0_body,
    grid=(_N // _RB,),
    in_specs=[
        pl.BlockSpec((_NCORES, _RB, 128), lambda i: (0, i, 0)),
        pl.BlockSpec((_NCORES, _RB, 16), lambda i: (0, i, 0)),
        pl.BlockSpec((_RB, 256), lambda i: (i, 0)),
        pl.BlockSpec((256, 256), lambda i: (0, 0)),
        pl.BlockSpec((256, 256), lambda i: (0, 0)),
        pl.BlockSpec((1, 256), lambda i: (0, 0)),
        pl.BlockSpec((256, 48), lambda i: (0, 0)),
        pl.BlockSpec((256, 48), lambda i: (0, 0)),
    ],
    out_specs=[
        pl.BlockSpec((_RB, 48), lambda i: (i, 0)),
        pl.BlockSpec((_RB, 48), lambda i: (i, 0)),
    ],
    out_shape=[
        jax.ShapeDtypeStruct((_N, 48), jnp.float32),
        jax.ShapeDtypeStruct((_N, 48), jnp.float32),
    ],
)


def _dense1_body(msg_ref, cnt_ref, y2_ref, c1_ref, out_ref):
  ssum = msg_ref[0] + msg_ref[1]
  cnt = jnp.max(cnt_ref[0] + cnt_ref[1], axis=1, keepdims=True)
  inv = 1.0 / jnp.maximum(cnt, 1.0)
  res = ssum * inv + y2_ref[...] + c1_ref[...]
  out_ref[...] = res[:, :40]


_dense1 = pl.pallas_call(
    _dense1_body,
    grid=(_N // _RB,),
    in_specs=[
        pl.BlockSpec((_NCORES, _RB, 48), lambda i: (0, i, 0)),
        pl.BlockSpec((_NCORES, _RB, 16), lambda i: (0, i, 0)),
        pl.BlockSpec((_RB, 48), lambda i: (i, 0)),
        pl.BlockSpec((1, 48), lambda i: (0, 0)),
    ],
    out_specs=pl.BlockSpec((_RB, 40), lambda i: (i, 0)),
    out_shape=jax.ShapeDtypeStruct((_N, 40), jnp.float32),
)


def kernel(x, edge_index, sage0_Wl, sage0_bl, sage0_Wr, lin0_W, lin0_b,
           sage1_Wl, sage1_bl, sage1_Wr, lin1_W, lin1_b):
  src = edge_index[0].astype(jnp.int32)
  dst = edge_index[1].astype(jnp.int32)

  x_lo = x[:, :128]
  x_hi = x[:, 128:]
  srcA = src.reshape(_NTILES, _NCH0, _CH0)
  dstA = dst.reshape(_NTILES, _NCH0, _CH0)
  zeros128 = jnp.zeros((_N // _NTILES, 128), jnp.float32)
  zeros16 = jnp.zeros((_N // _NTILES, 16), jnp.float32)
  ones = jnp.ones((_CH0, 16), jnp.float32)
  msg0, cnt = _segsum0(x_lo, x_hi, srcA, dstA, zeros128, zeros16, ones)

  b0 = (sage0_bl + lin0_b).reshape(1, 256)
  w2a = jnp.pad(sage1_Wl.T, ((0, 0), (0, 8)))
  w2b = jnp.pad(sage1_Wr.T, ((0, 0), (0, 8)))
  y1, y2 = _dense0(msg0, cnt, x, sage0_Wl.T, sage0_Wr.T, b0, w2a, w2b)

  srcC = src.reshape(_NCORES, _NTILES, _NCH1, _CH1)
  dstC = dst.reshape(_NCORES, _NTILES, _NCH1, _CH1)
  zeros48 = jnp.zeros((_N // _NTILES, 48), jnp.float32)
  (msg1,) = _segsum1(y1, srcC, dstC, zeros48)

  c1 = (sage1_bl + lin1_W @ lin0_b + lin1_b)
  c1p = jnp.pad(c1, (0, 8)).reshape(1, 48)
  return _dense1(msg1, cnt, y2, c1p)


# speedup vs baseline: 2.1585x; 1.0191x over previous
import jax
import jax.numpy as jnp
from jax import lax
from jax.experimental import pallas as pl
from jax.experimental.pallas import tpu as pltpu
from jax.experimental.pallas import tpu_sc as plsc

_N = 10000
_E = 160000
_NCORES = 2
_NTILES = 16
_OPT = _N // _NTILES


def _make_segsum(width, nch, ch, nbuf, nacc, with_counts, split_edges,
                 two_tables):
  ipt = nacc // _NTILES
  out_types = [jax.ShapeDtypeStruct((_NCORES, _N, width), jnp.float32)]
  scratch = [
      pltpu.VMEM_SHARED((nacc, width), jnp.float32),
      pltpu.VMEM((nch, ch), jnp.int32),
      pltpu.VMEM((nch, ch), jnp.int32),
  ] + [pltpu.VMEM((ch, width), jnp.float32) for _ in range(nbuf)] + [
      pltpu.SemaphoreType.DMA for _ in range(nbuf)
  ]
  if with_counts:
    out_types.append(jax.ShapeDtypeStruct((_NCORES, _N, 16), jnp.float32))
    scratch += [
        pltpu.VMEM_SHARED((nacc, 16), jnp.float32),
        pltpu.VMEM((ch, 16), jnp.float32),
    ]
  mesh = plsc.VectorSubcoreMesh(core_axis_name="c", subcore_axis_name="s")
  n_tables = 2 if two_tables else 1

  def body(*refs):
    it = iter(refs)
    tables = [next(it) for _ in range(n_tables)]
    src_hbm = next(it)
    dst_hbm = next(it)
    zeros_hbm = next(it)
    if with_counts:
      zcnt_hbm = next(it)
      ones_hbm = next(it)
    msg_hbm = next(it)
    if with_counts:
      cnt_hbm = next(it)
    acc_sh = next(it)
    src_v = next(it)
    dst_v = next(it)
    rows = tuple(next(it) for _ in range(nbuf))
    sems = tuple(next(it) for _ in range(nbuf))
    if with_counts:
      cnt_sh = next(it)
      ones_v = next(it)

    c = lax.axis_index("c")
    s = lax.axis_index("s")

    pltpu.sync_copy(zeros_hbm, acc_sh.at[pl.ds(s * ipt, ipt)])
    if with_counts:
      pltpu.sync_copy(zcnt_hbm, cnt_sh.at[pl.ds(s * ipt, ipt)])
      pltpu.sync_copy(ones_hbm, ones_v)
    if split_edges:
      pltpu.sync_copy(src_hbm.at[c, s], src_v)
      pltpu.sync_copy(dst_hbm.at[c, s], dst_v)
    else:
      pltpu.sync_copy(src_hbm.at[s], src_v)
      pltpu.sync_copy(dst_hbm.at[s], dst_v)
    plsc.subcore_barrier()

    def start_gather(j, b):
      if two_tables:
        @pl.when(c == 0)
        def _():
          pltpu.async_copy(tables[0].at[src_v.at[j]], rows[b], sems[b])

        @pl.when(c == 1)
        def _():
          pltpu.async_copy(tables[1].at[src_v.at[j]], rows[b], sems[b])
      else:
        pltpu.async_copy(tables[0].at[src_v.at[j]], rows[b], sems[b])

    def consume(j, b):
      pltpu.make_async_copy(tables[0].at[src_v.at[j]], rows[b],
                            sems[b]).wait()
      nxt = j + nbuf

      @pl.when(nxt < nch)
      def _():
        start_gather(nxt, b)

      pltpu.sync_copy(rows[b], acc_sh.at[dst_v.at[j]], add=True)
      if with_counts:
        @pl.when(((c == 0) & (j < nch // 2)) | ((c == 1) & (j >= nch // 2)))
        def _():
          pltpu.sync_copy(ones_v, cnt_sh.at[dst_v.at[j]], add=True)

    for b in range(min(nbuf, nch)):
      start_gather(b, b)
    main = nch - nch % nbuf

    @pl.loop(0, main, step=nbuf)
    def _(k):
      for b in range(nbuf):
        consume(k + b, b)

    for r in range(nch % nbuf):
      consume(main + r, r)

    plsc.subcore_barrier()
    pltpu.sync_copy(acc_sh.at[pl.ds(s * _OPT, _OPT)],
                    msg_hbm.at[c, pl.ds(s * _OPT, _OPT)])
    if with_counts:
      pltpu.sync_copy(cnt_sh.at[pl.ds(s * _OPT, _OPT)],
                      cnt_hbm.at[c, pl.ds(s * _OPT, _OPT)])

  return pl.kernel(body, out_type=tuple(out_types), mesh=mesh,
                   scratch_types=scratch,
                   compiler_params=pltpu.CompilerParams(
                       use_tc_tiling_on_sc=False))


_NCH0, _CH0 = 250, 40
_NCH1, _CH1 = 125, 40
_segsum0 = _make_segsum(width=128, nch=_NCH0, ch=_CH0, nbuf=3, nacc=_N,
                        with_counts=True, split_edges=False, two_tables=True)
_segsum1 = _make_segsum(width=48, nch=_NCH1, ch=_CH1, nbuf=6, nacc=_N,
                        with_counts=False, split_edges=True, two_tables=False)

_RB = 1000


def _dense0_body(msg_ref, cnt_ref, x_ref, wa_ref, wr_ref, b0_ref,
                 w2a_ref, w2b_ref, y1_ref, y2_ref):
  acc = jnp.dot(msg_ref[0], wa_ref[:128, :], preferred_element_type=jnp.float32)
  acc = acc + jnp.dot(msg_ref[1], wa_ref[128:, :],
                      preferred_element_type=jnp.float32)
  cnt = jnp.max(cnt_ref[0] + cnt_ref[1], axis=1, keepdims=True)
  inv = 1.0 / jnp.maximum(cnt, 1.0)
  h = acc * inv + jnp.dot(x_ref[...], wr_ref[...],
                          preferred_element_type=jnp.float32) + b0_ref[...]
  x1 = jnp.maximum(h, 0.0)
  y1_ref[...] = jnp.dot(x1, w2a_ref[...], preferred_element_type=jnp.float32)
  y2_ref[...] = jnp.dot(x1, w2b_ref[...], preferred_element_type=jnp.float32)


_dense0 = pl.pallas_call(
    _dense0_body,
    grid=(_N // _RB,),
    in_specs=[
        pl.BlockSpec((_NCORES, _RB, 128), lambda i: (0, i, 0)),
        pl.BlockSpec((_NCORES, _RB, 16), lambda i: (0, i, 0)),
        pl.BlockSpec((_RB, 256), lambda i: (i, 0)),
        pl.BlockSpec((256, 256), lambda i: (0, 0)),
        pl.BlockSpec((256, 256), lambda i: (0, 0)),
        pl.BlockSpec((1, 256), lambda i: (0, 0)),
        pl.BlockSpec((256, 48), lambda i: (0, 0)),
        pl.BlockSpec((256, 48), lambda i: (0, 0)),
    ],
    out_specs=[
        pl.BlockSpec((_RB, 48), lambda i: (i, 0)),
        pl.BlockSpec((_RB, 48), lambda i: (i, 0)),
    ],
    out_shape=[
        jax.ShapeDtypeStruct((_N, 48), jnp.float32),
        jax.ShapeDtypeStruct((_N, 48), jnp.float32),
    ],
)


def _dense1_body(msg_ref, cnt_ref, y2_ref, c1_ref, out_ref):
  ssum = msg_ref[0] + msg_ref[1]
  cnt = jnp.max(cnt_ref[0] + cnt_ref[1], axis=1, keepdims=True)
  inv = 1.0 / jnp.maximum(cnt, 1.0)
  res = ssum * inv + y2_ref[...] + c1_ref[...]
  out_ref[...] = res[:, :40]


_dense1 = pl.pallas_call(
    _dense1_body,
    grid=(_N // _RB,),
    in_specs=[
        pl.BlockSpec((_NCORES, _RB, 48), lambda i: (0, i, 0)),
        pl.BlockSpec((_NCORES, _RB, 16), lambda i: (0, i, 0)),
        pl.BlockSpec((_RB, 48), lambda i: (i, 0)),
        pl.BlockSpec((1, 48), lambda i: (0, 0)),
    ],
    out_specs=pl.BlockSpec((_RB, 40), lambda i: (i, 0)),
    out_shape=jax.ShapeDtypeStruct((_N, 40), jnp.float32),
)


def kernel(x, edge_index, sage0_Wl, sage0_bl, sage0_Wr, lin0_W, lin0_b,
           sage1_Wl, sage1_bl, sage1_Wr, lin1_W, lin1_b):
  src = edge_index[0].astype(jnp.int32)
  dst = edge_index[1].astype(jnp.int32)

  x_lo = x[:, :128]
  x_hi = x[:, 128:]
  srcA = src.reshape(_NTILES, _NCH0, _CH0)
  dstA = dst.reshape(_NTILES, _NCH0, _CH0)
  zeros128 = jnp.zeros((_N // _NTILES, 128), jnp.float32)
  zeros16 = jnp.zeros((_N // _NTILES, 16), jnp.float32)
  ones = jnp.ones((_CH0, 16), jnp.float32)
  msg0, cnt = _segsum0(x_lo, x_hi, srcA, dstA, zeros128, zeros16, ones)

  b0 = (sage0_bl + lin0_b).reshape(1, 256)
  w2a = jnp.pad(sage1_Wl.T, ((0, 0), (0, 8)))
  w2b = jnp.pad(sage1_Wr.T, ((0, 0), (0, 8)))
  y1, y2 = _dense0(msg0, cnt, x, sage0_Wl.T, sage0_Wr.T, b0, w2a, w2b)

  srcC = src.reshape(_NCORES, _NTILES, _NCH1, _CH1)
  dstC = dst.reshape(_NCORES, _NTILES, _NCH1, _CH1)
  zeros48 = jnp.zeros((_N // _NTILES, 48), jnp.float32)
  (msg1,) = _segsum1(y1, srcC, dstC, zeros48)

  c1 = (sage1_bl + lin1_W @ lin0_b + lin1_b)
  c1p = jnp.pad(c1, (0, 8)).reshape(1, 48)
  return _dense1(msg1, cnt, y2, c1p)
